# Initial kernel scaffold; baseline (speedup 1.0000x reference)
import jax
import jax.numpy as jnp
from jax.experimental import pallas as pl


def kernel(pred_dist, pred_bboxes, anchor_points, target_bboxes, target_scores, target_scores_sum, fg_mask):
    raise NotImplementedError("write your pallas kernel here")


# speedup vs baseline: 4.9364x; 4.9364x over previous
import jax
import jax.numpy as jnp
from jax.experimental import pallas as pl
from jax.experimental.pallas import tpu as pltpu

NBANDS = 10
SC = 1400
HALF = 2
NQ = 2
GROUP = 8


def _body(tss_ref, *refs):
    scores_refs = refs[0:NQ]
    pb_refs = refs[NQ:2 * NQ]
    tb_refs = refs[2 * NQ:3 * NQ]
    m_refs = refs[3 * NQ:4 * NQ]
    out_ref = refs[4 * NQ]
    (wt_s, y_s, tw_s, th_s, pw_s, ph_s, m_s,
     lc_s, ls_s, ymin_s, ymax_s, cnt_s, sw_s, sww_s, sh_s, shh_s) = \
        refs[4 * NQ + 1:]
    k = pl.program_id(0)
    nc = wt_s.shape[0]
    sub = wt_s.shape[1]
    p0 = nc // NQ
    p12 = nc // GROUP

    @pl.when(k == 0)
    def _init():
        lc_s[0] = 0.0
        ls_s[0] = 0.0
        ymin_s[0] = jnp.inf
        ymax_s[0] = -jnp.inf
        for b in range(NBANDS):
            cnt_s[b] = 0.0
            sw_s[b] = 0.0
            sww_s[b] = 0.0
            sh_s[b] = 0.0
            shh_s[b] = 0.0

    @pl.when(k < p0)
    def _phase0():
        ones = jnp.ones((scores_refs[0].shape[2], 8), jnp.float32)
        for q in range(NQ):
            g = k + q * p0
            for j in range(sub):
                sl = slice(j * SC, (j + 1) * SC)
                m = m_refs[q][0, 0, sl]
                ws8 = jax.lax.dot_general(
                    scores_refs[q][0, sl, :], ones, (((1,), (0,)), ((), ())),
                    preferred_element_type=jnp.float32)
                ws = jnp.swapaxes(ws8, 0, 1)[0]
                wt = ws * m

                pbb = jnp.swapaxes(pb_refs[q][0, sl, :], 0, 1)
                tbb = jnp.swapaxes(tb_refs[q][0, sl, :], 0, 1)
                pw = pbb[2] - pbb[0]
                phh = pbb[3] - pbb[1]
                pcx = pbb[0] + pw * 0.5
                pcy = pbb[1] + phh * 0.5
                tw = tbb[2] - tbb[0]
                th = tbb[3] - tbb[1]
                tcx = tbb[0] + tw * 0.5
                tcy = tbb[1] + th * 0.5

                cd2 = (pcx - tcx) ** 2 + (pcy - tcy) ** 2
                lc_s[0] += jnp.sum(cd2 * wt)
                ymin_s[0] = jnp.minimum(
                    ymin_s[0], jnp.min(jnp.where(m > 0, tcy, jnp.inf)))
                ymax_s[0] = jnp.maximum(
                    ymax_s[0], jnp.max(jnp.where(m > 0, tcy, -jnp.inf)))

                wt_s[g, j, :] = wt
                y_s[g, j, :] = tcy
                tw_s[g, j, :] = tw
                th_s[g, j, :] = th
                pw_s[g, j, :] = pw
                ph_s[g, j, :] = phh
                m_s[g, j, :] = m

    def _bands(rows, j):
        y = y_s[rows, j, :]
        rng = ymax_s[0] - ymin_s[0]
        ynorm = (y - ymin_s[0]) / (rng + 1e-6)
        band = jnp.clip((ynorm * NBANDS).astype(jnp.int32), 0, NBANDS - 1)
        return jnp.where(rng < 1e-6, jnp.zeros_like(band), band)

    @pl.when(jnp.logical_and(k >= p0, k < p0 + p12))
    def _phase1():
        rows = pl.ds((k - p0) * GROUP, GROUP)
        for j in range(sub):
            band = _bands(rows, j)
            m = m_s[rows, j, :]
            tw = tw_s[rows, j, :]
            th = th_s[rows, j, :]
            tw2 = tw * tw
            th2 = th * th
            for b in range(NBANDS):
                sel = jnp.where(band == b, m, 0.0)
                cnt_s[b] += jnp.sum(sel)
                sw_s[b] += jnp.sum(tw * sel)
                sww_s[b] += jnp.sum(tw2 * sel)
                sh_s[b] += jnp.sum(th * sel)
                shh_s[b] += jnp.sum(th2 * sel)

    @pl.when(k >= p0 + p12)
    def _phase2():
        rows = pl.ds((k - p0 - p12) * GROUP, GROUP)
        for j in range(sub):
            band = _bands(rows, j)
            shape = band.shape
            cnt_e = jnp.zeros(shape, jnp.float32)
            sw_e = jnp.zeros(shape, jnp.float32)
            sww_e = jnp.zeros(shape, jnp.float32)
            sh_e = jnp.zeros(shape, jnp.float32)
            shh_e = jnp.zeros(shape, jnp.float32)
            for b in range(NBANDS):
                hit = band == b
                cnt_e = jnp.where(hit, cnt_s[b], cnt_e)
                sw_e = jnp.where(hit, sw_s[b], sw_e)
                sww_e = jnp.where(hit, sww_s[b], sww_e)
                sh_e = jnp.where(hit, sh_s[b], sh_e)
                shh_e = jnp.where(hit, shh_s[b], shh_e)

            cmax = jnp.maximum(cnt_e, 1.0)
            cm1 = jnp.maximum(cnt_e - 1.0, 1.0)
            mean_w = sw_e / cmax
            mean_h = sh_e / cmax
            std_w = jnp.sqrt(jnp.maximum(
                (sww_e - cnt_e * mean_w * mean_w) / cm1, 0.0)) + 1e-6
            std_h = jnp.sqrt(jnp.maximum(
                (shh_e - cnt_e * mean_h * mean_h) / cm1, 0.0)) + 1e-6
            ub_w = mean_w + 3.0 * std_w
            lb_w = jnp.maximum(mean_w - 3.0 * std_w, 0.0)
            ub_h = mean_h + 3.0 * std_h
            lb_h = jnp.maximum(mean_h - 3.0 * std_h, 0.0)

            pw = pw_s[rows, j, :]
            phh = ph_s[rows, j, :]
            lw = jnp.where(pw > ub_w, (pw - ub_w) ** 2,
                           jnp.where(pw < lb_w, (lb_w - pw) ** 2, 0.0))
            lh = jnp.where(phh > ub_h, (phh - ub_h) ** 2,
                           jnp.where(phh < lb_h, (lb_h - phh) ** 2, 0.0))
            loss = jnp.where(cnt_e >= 2.0, lw + lh, 0.0)
            ls_s[0] += jnp.sum(loss * wt_s[rows, j, :])

        @pl.when(k == pl.num_programs(0) - 1)
        def _finish():
            out_ref[0] = (lc_s[0] + ls_s[0]) / tss_ref[0]


def kernel(pred_dist, pred_bboxes, anchor_points, target_bboxes,
           target_scores, target_scores_sum, fg_mask):
    B, A, C = target_scores.shape
    chunk = A // HALF
    sub = chunk // SC
    nc = B * HALF
    p0 = nc // NQ
    p12 = nc // GROUP
    m32 = fg_mask.astype(jnp.float32).reshape(nc, 1, chunk)
    tss = target_scores_sum.reshape(1)

    def _map3(q):
        def _map(k):
            c = jnp.minimum(k, p0 - 1) + q * p0
            return (c // HALF, c % HALF, 0)
        return _map

    def _mapm(q):
        def _map(k):
            return (jnp.minimum(k, p0 - 1) + q * p0, 0, 0)
        return _map

    score_specs = [pl.BlockSpec((1, chunk, C), _map3(q)) for q in range(NQ)]
    box_specs = [pl.BlockSpec((1, chunk, 4), _map3(q)) for q in range(NQ)]
    m_specs = [pl.BlockSpec((1, 1, chunk), _mapm(q)) for q in range(NQ)]

    loss = pl.pallas_call(
        _body,
        grid=(p0 + 2 * p12,),
        in_specs=([pl.BlockSpec(memory_space=pltpu.SMEM)]
                  + score_specs + box_specs + box_specs + m_specs),
        out_specs=pl.BlockSpec(memory_space=pltpu.SMEM),
        out_shape=jax.ShapeDtypeStruct((1,), jnp.float32),
        scratch_shapes=[
            pltpu.VMEM((nc, sub, SC), jnp.float32),
            pltpu.VMEM((nc, sub, SC), jnp.float32),
            pltpu.VMEM((nc, sub, SC), jnp.float32),
            pltpu.VMEM((nc, sub, SC), jnp.float32),
            pltpu.VMEM((nc, sub, SC), jnp.float32),
            pltpu.VMEM((nc, sub, SC), jnp.float32),
            pltpu.VMEM((nc, sub, SC), jnp.float32),
            pltpu.SMEM((1,), jnp.float32),
            pltpu.SMEM((1,), jnp.float32),
            pltpu.SMEM((1,), jnp.float32),
            pltpu.SMEM((1,), jnp.float32),
            pltpu.SMEM((NBANDS,), jnp.float32),
            pltpu.SMEM((NBANDS,), jnp.float32),
            pltpu.SMEM((NBANDS,), jnp.float32),
            pltpu.SMEM((NBANDS,), jnp.float32),
            pltpu.SMEM((NBANDS,), jnp.float32),
        ],
        interpret=False,
    )(tss,
      *([target_scores] * NQ), *([pred_bboxes] * NQ),
      *([target_bboxes] * NQ), *([m32] * NQ))

    loss_bbox = loss[0].reshape(())
    loss_dfl = jnp.zeros((), dtype=jnp.float32)
    return (loss_bbox, loss_dfl)
